# Initial kernel scaffold; baseline (speedup 1.0000x reference)
import jax
import jax.numpy as jnp
from jax.experimental import pallas as pl


def kernel(x, edge_index, W1, b1, W2, b2, beta2):
    raise NotImplementedError("write your pallas kernel here")


# speedup vs baseline: 16.6504x; 16.6504x over previous
import functools

import jax
import jax.numpy as jnp
from jax import lax
from jax.experimental import pallas as pl
from jax.experimental.pallas import tpu as pltpu
from jax.experimental.pallas import tpu_sc as plsc

N = 10000
E = 320000
D = 128
H = 16
C = 40

NC = 2
NS = 16
NW = NC * NS
EPW = E // NW
EB = 80
NB = EPW // EB
GW = 32
ZR = 624
ZTAIL = N - NS * ZR

_ROWBLK = 1000


def _tc_in_body(x_ref, w1t_ref, b1_ref, g_ref):
    h = jnp.dot(x_ref[...], w1t_ref[...], preferred_element_type=jnp.float32)
    h = jnp.maximum(h + b1_ref[...], 0.0)
    n2 = jnp.sum(h * h, axis=1, keepdims=True)
    rnorm = 1.0 / jnp.maximum(jnp.sqrt(n2), 1e-12)
    c_self = n2 * rnorm * rnorm
    es = jnp.exp(c_self - 1.0)
    ones = jnp.ones_like(n2)
    zeros = jnp.zeros((h.shape[0], GW - H - 3), jnp.float32)
    g_ref[...] = jnp.concatenate([h, ones, rnorm, es, zeros], axis=1)


def _tc_mid_body(pa_ref, pb_ref, g_ref, beta_ref, gout_ref):
    beta = beta_ref[0, 0]
    pa = pa_ref[...]
    pb = pb_ref[...]
    g = g_ref[...]
    h_prev = g[:, 0:H]
    es_prev = g[:, H + 2:H + 3]
    num = pa[:, 0:H] + pb[:, 0:H] + h_prev * es_prev
    den = pa[:, H:H + 1] + pb[:, H:H + 1] + es_prev
    h = num / den
    n2 = jnp.sum(h * h, axis=1, keepdims=True)
    rnorm = 1.0 / jnp.maximum(jnp.sqrt(n2), 1e-12)
    c_self = n2 * rnorm * rnorm
    es = jnp.exp(beta * c_self - jnp.abs(beta))
    ones = jnp.ones_like(n2)
    zeros = jnp.zeros((h.shape[0], GW - H - 3), jnp.float32)
    gout_ref[...] = jnp.concatenate([h, ones, rnorm, es, zeros], axis=1)


def _tc_out_body(pa_ref, pb_ref, g_ref, w2t_ref, b2_ref, out_ref):
    pa = pa_ref[...]
    pb = pb_ref[...]
    g = g_ref[...]
    h_prev = g[:, 0:H]
    es_prev = g[:, H + 2:H + 3]
    num = pa[:, 0:H] + pb[:, 0:H] + h_prev * es_prev
    den = pa[:, H:H + 1] + pb[:, H:H + 1] + es_prev
    h = num / den
    logits = jnp.dot(h, w2t_ref[...], preferred_element_type=jnp.float32)
    logits = logits + b2_ref[...]
    m = jnp.max(logits, axis=1, keepdims=True)
    lse = jnp.log(jnp.sum(jnp.exp(logits - m), axis=1, keepdims=True)) + m
    out_ref[...] = logits - lse


def _tc_in(x, w1t, b1):
    return pl.pallas_call(
        _tc_in_body,
        grid=(N // _ROWBLK,),
        in_specs=[
            pl.BlockSpec((_ROWBLK, D), lambda i: (i, 0)),
            pl.BlockSpec((D, H), lambda i: (0, 0)),
            pl.BlockSpec((1, H), lambda i: (0, 0)),
        ],
        out_specs=pl.BlockSpec((_ROWBLK, GW), lambda i: (i, 0)),
        out_shape=jax.ShapeDtypeStruct((N, GW), jnp.float32),
    )(x, w1t, b1)


def _tc_mid(pa, pb, g, beta):
    return pl.pallas_call(
        _tc_mid_body,
        grid=(N // _ROWBLK,),
        in_specs=[
            pl.BlockSpec((_ROWBLK, GW), lambda i: (i, 0)),
            pl.BlockSpec((_ROWBLK, GW), lambda i: (i, 0)),
            pl.BlockSpec((_ROWBLK, GW), lambda i: (i, 0)),
            pl.BlockSpec((1, 1), lambda i: (0, 0)),
        ],
        out_specs=pl.BlockSpec((_ROWBLK, GW), lambda i: (i, 0)),
        out_shape=jax.ShapeDtypeStruct((N, GW), jnp.float32),
    )(pa, pb, g, beta)


def _tc_out(pa, pb, g, w2t, b2):
    return pl.pallas_call(
        _tc_out_body,
        grid=(N // _ROWBLK,),
        in_specs=[
            pl.BlockSpec((_ROWBLK, GW), lambda i: (i, 0)),
            pl.BlockSpec((_ROWBLK, GW), lambda i: (i, 0)),
            pl.BlockSpec((_ROWBLK, GW), lambda i: (i, 0)),
            pl.BlockSpec((H, C), lambda i: (0, 0)),
            pl.BlockSpec((1, C), lambda i: (0, 0)),
        ],
        out_specs=pl.BlockSpec((_ROWBLK, C), lambda i: (i, 0)),
        out_shape=jax.ShapeDtypeStruct((N, C), jnp.float32),
    )(pa, pb, g, w2t, b2)


def _sc_prop(g, srcs, dsts, bvec):
    mesh = plsc.VectorSubcoreMesh(
        core_axis_name="c", subcore_axis_name="s",
        num_cores=NC, num_subcores=NS)

    @functools.partial(
        pl.kernel,
        out_type=jax.ShapeDtypeStruct((NC, N, GW), jnp.float32),
        mesh=mesh,
        scratch_types=[
            pltpu.VMEM((EB,), jnp.int32),
            pltpu.VMEM((EB,), jnp.int32),
            pltpu.VMEM((EB, GW), jnp.float32),
            pltpu.VMEM((EB, GW), jnp.float32),
            pltpu.VMEM((EB, GW), jnp.float32),
            pltpu.VMEM((16,), jnp.float32),
            pltpu.VMEM((ZR, GW), jnp.float32),
            pltpu.VMEM_SHARED((N, GW), jnp.float32),
            pltpu.SemaphoreType.DMA,
            pltpu.SemaphoreType.DMA,
        ],
        compiler_params=pltpu.CompilerParams(
            needs_layout_passes=False, use_tc_tiling_on_sc=False),
    )
    def k(g_hbm, src_hbm, dst_hbm, bvec_hbm, out_hbm,
          src_v, dst_v, gs_v, gd_v, ct_v, b_v, z_v, acc_sh, sem1, sem2):
        cid = lax.axis_index("c")
        sid = lax.axis_index("s")
        wid = cid * NS + sid

        zero16 = jnp.zeros((16,), jnp.float32)

        def zbody(i, carry):
            z_v[i, 0:16] = zero16
            z_v[i, 16:32] = zero16
            return carry

        lax.fori_loop(0, ZR, zbody, 0)
        pltpu.sync_copy(z_v, acc_sh.at[pl.ds(sid * ZR, ZR)])

        @pl.when(sid == NS - 1)
        def _():
            pltpu.sync_copy(z_v.at[pl.ds(0, ZTAIL)],
                            acc_sh.at[pl.ds(NS * ZR, ZTAIL)])

        pltpu.sync_copy(bvec_hbm, b_v)
        plsc.subcore_barrier()

        bv16 = b_v[...]
        beta = bv16[0]
        ab = bv16[1]
        ebase = wid * EPW
        col17 = jnp.full((16,), H + 1, jnp.int32)

        def blk(bi, carry):
            base = ebase + bi * EB
            pltpu.sync_copy(src_hbm.at[pl.ds(base, EB)], src_v)
            pltpu.sync_copy(dst_hbm.at[pl.ds(base, EB)], dst_v)
            cp1 = pltpu.async_copy(g_hbm.at[src_v], gs_v, sem1)
            cp2 = pltpu.async_copy(g_hbm.at[dst_v], gd_v, sem2)
            cp1.wait()
            cp2.wait()
            for grp in range(EB // 16):
                rows = lax.iota(jnp.int32, 16) + (grp * 16)
                score = jnp.zeros((16,), jnp.float32)
                for kcol in range(H):
                    colv = jnp.full((16,), kcol, jnp.int32)
                    a = plsc.load_gather(gs_v, [rows, colv])
                    b = plsc.load_gather(gd_v, [rows, colv])
                    score = score + a * b
                rn_s = plsc.load_gather(gs_v, [rows, col17])
                rn_d = plsc.load_gather(gd_v, [rows, col17])
                e16 = jnp.exp(beta * score * rn_s * rn_d - ab)
                for jj in range(16):
                    j = grp * 16 + jj
                    ev = e16[jj]
                    ct_v[j, 0:16] = gs_v[j, 0:16] * ev
                    ct_v[j, 16:32] = gs_v[j, 16:32] * ev
            pltpu.sync_copy(ct_v, acc_sh.at[dst_v], add=True)
            return carry

        lax.fori_loop(0, NB, blk, 0)
        plsc.subcore_barrier()
        pltpu.sync_copy(acc_sh.at[pl.ds(sid * ZR, ZR)],
                        out_hbm.at[cid, pl.ds(sid * ZR, ZR)])

        @pl.when(sid == NS - 1)
        def _():
            pltpu.sync_copy(acc_sh.at[pl.ds(NS * ZR, ZTAIL)],
                            out_hbm.at[cid, pl.ds(NS * ZR, ZTAIL)])

    return k(g, srcs, dsts, bvec)


def kernel(x, edge_index, W1, b1, W2, b2, beta2):
    x = x.astype(jnp.float32)
    ei = edge_index.astype(jnp.int32)
    srcs = ei[0]
    dsts = ei[1]
    w1t = W1.astype(jnp.float32).T
    w2t = W2.astype(jnp.float32).T
    b1r = b1.astype(jnp.float32).reshape(1, H)
    b2r = b2.astype(jnp.float32).reshape(1, C)
    beta = beta2.astype(jnp.float32)[0]

    g0 = _tc_in(x, w1t, b1r)
    bvec1 = jnp.zeros((16,), jnp.float32).at[0].set(1.0).at[1].set(1.0)
    parts1 = _sc_prop(g0, srcs, dsts, bvec1)
    g1 = _tc_mid(parts1[0], parts1[1], g0, beta2.astype(jnp.float32).reshape(1, 1))
    bvec2 = jnp.zeros((16,), jnp.float32).at[0].set(beta).at[1].set(jnp.abs(beta))
    parts2 = _sc_prop(g1, srcs, dsts, bvec2)
    return _tc_out(parts2[0], parts2[1], g1, w2t, b2r)


# speedup vs baseline: 40.8175x; 2.4514x over previous
import functools

import jax
import jax.numpy as jnp
from jax import lax
from jax.experimental import pallas as pl
from jax.experimental.pallas import tpu as pltpu
from jax.experimental.pallas import tpu_sc as plsc

N = 10000
E = 320000
D = 128
H = 16
C = 40

NC = 2
NS = 16
NW = NC * NS
EPW = E // NW
EB = 80
NB = EPW // EB
GW = 32
ZR = 624
ZTAIL = N - NS * ZR

_ROWBLK = 1000


def _tc_in_body(x_ref, w1t_ref, b1_ref, xn_ref, nrm_ref):
    h = jnp.dot(x_ref[...], w1t_ref[...], preferred_element_type=jnp.float32)
    h = jnp.maximum(h + b1_ref[...], 0.0)
    n2 = jnp.sum(h * h, axis=1, keepdims=True)
    norm = jnp.sqrt(n2)
    rnorm = 1.0 / jnp.maximum(norm, 1e-12)
    xn_ref[...] = h * rnorm
    nrm_ref[...] = norm


def _tc_mid_body(pa_ref, pb_ref, xn_ref, nrm_ref, xno_ref, nrmo_ref):
    pa = pa_ref[...]
    pb = pb_ref[...]
    xn = xn_ref[...]
    h_prev = xn * nrm_ref[...]
    c_prev = jnp.sum(xn * xn, axis=1, keepdims=True)
    es = jnp.exp(c_prev - 1.0)
    num = pa[:, 0:H] + pb[:, 0:H] + h_prev * es
    den = pa[:, H:H + 1] + pb[:, H:H + 1] + es
    h = num / den
    n2 = jnp.sum(h * h, axis=1, keepdims=True)
    norm = jnp.sqrt(n2)
    rnorm = 1.0 / jnp.maximum(norm, 1e-12)
    xno_ref[...] = h * rnorm
    nrmo_ref[...] = norm


def _tc_out_body(pa_ref, pb_ref, xn_ref, nrm_ref, beta_ref, w2t_ref, b2_ref,
                 out_ref):
    beta = beta_ref[0, 0]
    pa = pa_ref[...]
    pb = pb_ref[...]
    xn = xn_ref[...]
    h_prev = xn * nrm_ref[...]
    c_prev = jnp.sum(xn * xn, axis=1, keepdims=True)
    es = jnp.exp(beta * c_prev - jnp.abs(beta))
    num = pa[:, 0:H] + pb[:, 0:H] + h_prev * es
    den = pa[:, H:H + 1] + pb[:, H:H + 1] + es
    h = num / den
    logits = jnp.dot(h, w2t_ref[...], preferred_element_type=jnp.float32)
    logits = logits + b2_ref[...]
    m = jnp.max(logits, axis=1, keepdims=True)
    lse = jnp.log(jnp.sum(jnp.exp(logits - m), axis=1, keepdims=True)) + m
    out_ref[...] = logits - lse


def _tc_in(x, w1t, b1):
    return pl.pallas_call(
        _tc_in_body,
        grid=(N // _ROWBLK,),
        in_specs=[
            pl.BlockSpec((_ROWBLK, D), lambda i: (i, 0)),
            pl.BlockSpec((D, H), lambda i: (0, 0)),
            pl.BlockSpec((1, H), lambda i: (0, 0)),
        ],
        out_specs=[
            pl.BlockSpec((_ROWBLK, H), lambda i: (i, 0)),
            pl.BlockSpec((_ROWBLK, 1), lambda i: (i, 0)),
        ],
        out_shape=[
            jax.ShapeDtypeStruct((N, H), jnp.float32),
            jax.ShapeDtypeStruct((N, 1), jnp.float32),
        ],
    )(x, w1t, b1)


def _tc_mid(pa, pb, xn, nrm):
    return pl.pallas_call(
        _tc_mid_body,
        grid=(N // _ROWBLK,),
        in_specs=[
            pl.BlockSpec((_ROWBLK, GW), lambda i: (i, 0)),
            pl.BlockSpec((_ROWBLK, GW), lambda i: (i, 0)),
            pl.BlockSpec((_ROWBLK, H), lambda i: (i, 0)),
            pl.BlockSpec((_ROWBLK, 1), lambda i: (i, 0)),
        ],
        out_specs=[
            pl.BlockSpec((_ROWBLK, H), lambda i: (i, 0)),
            pl.BlockSpec((_ROWBLK, 1), lambda i: (i, 0)),
        ],
        out_shape=[
            jax.ShapeDtypeStruct((N, H), jnp.float32),
            jax.ShapeDtypeStruct((N, 1), jnp.float32),
        ],
    )(pa, pb, xn, nrm)


def _tc_out(pa, pb, xn, nrm, beta, w2t, b2):
    return pl.pallas_call(
        _tc_out_body,
        grid=(N // _ROWBLK,),
        in_specs=[
            pl.BlockSpec((_ROWBLK, GW), lambda i: (i, 0)),
            pl.BlockSpec((_ROWBLK, GW), lambda i: (i, 0)),
            pl.BlockSpec((_ROWBLK, H), lambda i: (i, 0)),
            pl.BlockSpec((_ROWBLK, 1), lambda i: (i, 0)),
            pl.BlockSpec((1, 1), lambda i: (0, 0)),
            pl.BlockSpec((H, C), lambda i: (0, 0)),
            pl.BlockSpec((1, C), lambda i: (0, 0)),
        ],
        out_specs=pl.BlockSpec((_ROWBLK, C), lambda i: (i, 0)),
        out_shape=jax.ShapeDtypeStruct((N, C), jnp.float32),
    )(pa, pb, xn, nrm, beta, w2t, b2)


def _sc_prop(xn, nrm, src2d, dst2d, bvec):
    mesh = plsc.VectorSubcoreMesh(
        core_axis_name="c", subcore_axis_name="s",
        num_cores=NC, num_subcores=NS)

    @functools.partial(
        pl.kernel,
        out_type=jax.ShapeDtypeStruct((NC, N, GW), jnp.float32),
        mesh=mesh,
        scratch_types=[
            pltpu.VMEM((NB, EB), jnp.int32),
            pltpu.VMEM((NB, EB), jnp.int32),
            pltpu.VMEM((N,), jnp.float32),
            pltpu.VMEM((EB, H), jnp.float32),
            pltpu.VMEM((EB, H), jnp.float32),
            pltpu.VMEM((EB, H), jnp.float32),
            pltpu.VMEM((EB, H), jnp.float32),
            pltpu.VMEM((EB, GW), jnp.float32),
            pltpu.VMEM((16,), jnp.float32),
            pltpu.VMEM((ZR, GW), jnp.float32),
            pltpu.VMEM_SHARED((N, GW), jnp.float32),
            pltpu.SemaphoreType.DMA,
            pltpu.SemaphoreType.DMA,
            pltpu.SemaphoreType.DMA,
            pltpu.SemaphoreType.DMA,
        ],
        compiler_params=pltpu.CompilerParams(
            needs_layout_passes=False, use_tc_tiling_on_sc=False),
    )
    def k(xn_hbm, nrm_hbm, src_hbm, dst_hbm, bvec_hbm, out_hbm,
          src_v, dst_v, nrm_v, gs0, gs1, gd0, gd1, ct_v, b_v, z_v, acc_sh,
          sgs0, sgs1, sgd0, sgd1):
        cid = lax.axis_index("c")
        sid = lax.axis_index("s")
        wid = cid * NS + sid

        zero16 = jnp.zeros((16,), jnp.float32)

        def zbody(i, carry):
            z_v[i, 0:16] = zero16
            z_v[i, 16:32] = zero16
            return carry

        lax.fori_loop(0, ZR, zbody, 0)
        pltpu.sync_copy(z_v, acc_sh.at[pl.ds(sid * ZR, ZR)])

        @pl.when(sid == NS - 1)
        def _():
            pltpu.sync_copy(z_v.at[pl.ds(0, ZTAIL)],
                            acc_sh.at[pl.ds(NS * ZR, ZTAIL)])

        pltpu.sync_copy(bvec_hbm, b_v)
        pltpu.sync_copy(nrm_hbm, nrm_v)
        pltpu.sync_copy(src_hbm.at[pl.ds(wid * NB, NB)], src_v)
        pltpu.sync_copy(dst_hbm.at[pl.ds(wid * NB, NB)], dst_v)
        plsc.subcore_barrier()

        bv16 = b_v[...]
        beta = bv16[0]
        ab = bv16[1]
        onehot0 = (lax.iota(jnp.int32, 16) == 0).astype(jnp.float32)
        slots = ((gs0, gd0, sgs0, sgd0), (gs1, gd1, sgs1, sgd1))

        def issue(bi, slot):
            gs, gd, sgs, sgd = slots[slot]
            pltpu.async_copy(xn_hbm.at[src_v.at[bi]], gs, sgs)
            pltpu.async_copy(xn_hbm.at[dst_v.at[bi]], gd, sgd)

        def wait_slot(slot):
            gs, gd, sgs, sgd = slots[slot]
            pltpu.make_async_copy(xn_hbm.at[pl.ds(0, EB)], gs, sgs).wait()
            pltpu.make_async_copy(xn_hbm.at[pl.ds(0, EB)], gd, sgd).wait()

        def compute_scatter(bi, slot):
            gs, gd, _, _ = slots[slot]
            for grp in range(EB // 16):
                rows = lax.iota(jnp.int32, 16) + (grp * 16)
                score = jnp.zeros((16,), jnp.float32)
                for kcol in range(H):
                    colv = jnp.full((16,), kcol, jnp.int32)
                    a = plsc.load_gather(gs, [rows, colv])
                    b = plsc.load_gather(gd, [rows, colv])
                    score = score + a * b
                s16 = src_v[bi, pl.ds(grp * 16, 16)]
                nsrc = plsc.load_gather(nrm_v, [s16])
                e16 = jnp.exp(beta * score - ab)
                sc16 = e16 * nsrc
                for jj in range(16):
                    j = grp * 16 + jj
                    ct_v[j, 0:16] = gs[j, 0:16] * sc16[jj]
                    ct_v[j, 16:32] = onehot0 * e16[jj]
            pltpu.sync_copy(ct_v, acc_sh.at[dst_v.at[bi]], add=True)

        issue(0, 0)

        def pair(g, carry):
            b0 = 2 * g
            issue(b0 + 1, 1)
            wait_slot(0)
            compute_scatter(b0, 0)
            issue(b0 + 2, 0)
            wait_slot(1)
            compute_scatter(b0 + 1, 1)
            return carry

        lax.fori_loop(0, (NB - 1) // 2, pair, 0)
        wait_slot(0)
        compute_scatter(NB - 1, 0)

        plsc.subcore_barrier()
        pltpu.sync_copy(acc_sh.at[pl.ds(sid * ZR, ZR)],
                        out_hbm.at[cid, pl.ds(sid * ZR, ZR)])

        @pl.when(sid == NS - 1)
        def _():
            pltpu.sync_copy(acc_sh.at[pl.ds(NS * ZR, ZTAIL)],
                            out_hbm.at[cid, pl.ds(NS * ZR, ZTAIL)])

    return k(xn, nrm, src2d, dst2d, bvec)


def kernel(x, edge_index, W1, b1, W2, b2, beta2):
    x = x.astype(jnp.float32)
    ei = edge_index.astype(jnp.int32)
    src2d = ei[0].reshape(NW * NB, EB)
    dst2d = ei[1].reshape(NW * NB, EB)
    w1t = W1.astype(jnp.float32).T
    w2t = W2.astype(jnp.float32).T
    b1r = b1.astype(jnp.float32).reshape(1, H)
    b2r = b2.astype(jnp.float32).reshape(1, C)
    beta = beta2.astype(jnp.float32)[0]

    xn0, nrm0 = _tc_in(x, w1t, b1r)
    bvec1 = jnp.zeros((16,), jnp.float32).at[0].set(1.0).at[1].set(1.0)
    parts1 = _sc_prop(xn0, nrm0.reshape(N), src2d, dst2d, bvec1)
    xn1, nrm1 = _tc_mid(parts1[0], parts1[1], xn0, nrm0)
    bvec2 = jnp.zeros((16,), jnp.float32).at[0].set(beta).at[1].set(jnp.abs(beta))
    parts2 = _sc_prop(xn1, nrm1.reshape(N), src2d, dst2d, bvec2)
    return _tc_out(parts2[0], parts2[1], xn1, nrm1,
                   beta2.astype(jnp.float32).reshape(1, 1), w2t, b2r)


# speedup vs baseline: 44.2694x; 1.0846x over previous
import functools

import jax
import jax.numpy as jnp
from jax import lax
from jax.experimental import pallas as pl
from jax.experimental.pallas import tpu as pltpu
from jax.experimental.pallas import tpu_sc as plsc

N = 10000
E = 320000
D = 128
H = 16
C = 40

NC = 2
NS = 16
NW = NC * NS
EPW = E // NW
EB = 80
NB = EPW // EB
GW = 32
ZR = 624
ZTAIL = N - NS * ZR

_ROWBLK = 1000


def _tc_in_body(x_ref, w1t_ref, b1_ref, xn_ref, nrm_ref):
    h = jnp.dot(x_ref[...], w1t_ref[...], preferred_element_type=jnp.float32)
    h = jnp.maximum(h + b1_ref[...], 0.0)
    n2 = jnp.sum(h * h, axis=1, keepdims=True)
    norm = jnp.sqrt(n2)
    rnorm = 1.0 / jnp.maximum(norm, 1e-12)
    xn_ref[...] = h * rnorm
    nrm_ref[...] = norm


def _tc_mid_body(pa_ref, pb_ref, xn_ref, nrm_ref, xno_ref, nrmo_ref):
    pa = pa_ref[...]
    pb = pb_ref[...]
    xn = xn_ref[...]
    h_prev = xn * nrm_ref[...]
    c_prev = jnp.sum(xn * xn, axis=1, keepdims=True)
    es = jnp.exp(c_prev - 1.0)
    num = pa[:, 0:H] + pb[:, 0:H] + h_prev * es
    den = (jnp.sum(pa[:, H:GW], axis=1, keepdims=True)
           + jnp.sum(pb[:, H:GW], axis=1, keepdims=True) + es)
    h = num / den
    n2 = jnp.sum(h * h, axis=1, keepdims=True)
    norm = jnp.sqrt(n2)
    rnorm = 1.0 / jnp.maximum(norm, 1e-12)
    xno_ref[...] = h * rnorm
    nrmo_ref[...] = norm


def _tc_out_body(pa_ref, pb_ref, xn_ref, nrm_ref, beta_ref, w2t_ref, b2_ref,
                 out_ref):
    beta = beta_ref[0, 0]
    pa = pa_ref[...]
    pb = pb_ref[...]
    xn = xn_ref[...]
    h_prev = xn * nrm_ref[...]
    c_prev = jnp.sum(xn * xn, axis=1, keepdims=True)
    es = jnp.exp(beta * c_prev - jnp.abs(beta))
    num = pa[:, 0:H] + pb[:, 0:H] + h_prev * es
    den = (jnp.sum(pa[:, H:GW], axis=1, keepdims=True)
           + jnp.sum(pb[:, H:GW], axis=1, keepdims=True) + es)
    h = num / den
    logits = jnp.dot(h, w2t_ref[...], preferred_element_type=jnp.float32)
    logits = logits + b2_ref[...]
    m = jnp.max(logits, axis=1, keepdims=True)
    lse = jnp.log(jnp.sum(jnp.exp(logits - m), axis=1, keepdims=True)) + m
    out_ref[...] = logits - lse


def _tc_in(x, w1t, b1):
    return pl.pallas_call(
        _tc_in_body,
        grid=(N // _ROWBLK,),
        in_specs=[
            pl.BlockSpec((_ROWBLK, D), lambda i: (i, 0)),
            pl.BlockSpec((D, H), lambda i: (0, 0)),
            pl.BlockSpec((1, H), lambda i: (0, 0)),
        ],
        out_specs=[
            pl.BlockSpec((_ROWBLK, H), lambda i: (i, 0)),
            pl.BlockSpec((_ROWBLK, 1), lambda i: (i, 0)),
        ],
        out_shape=[
            jax.ShapeDtypeStruct((N, H), jnp.float32),
            jax.ShapeDtypeStruct((N, 1), jnp.float32),
        ],
    )(x, w1t, b1)


def _tc_mid(pa, pb, xn, nrm):
    return pl.pallas_call(
        _tc_mid_body,
        grid=(N // _ROWBLK,),
        in_specs=[
            pl.BlockSpec((_ROWBLK, GW), lambda i: (i, 0)),
            pl.BlockSpec((_ROWBLK, GW), lambda i: (i, 0)),
            pl.BlockSpec((_ROWBLK, H), lambda i: (i, 0)),
            pl.BlockSpec((_ROWBLK, 1), lambda i: (i, 0)),
        ],
        out_specs=[
            pl.BlockSpec((_ROWBLK, H), lambda i: (i, 0)),
            pl.BlockSpec((_ROWBLK, 1), lambda i: (i, 0)),
        ],
        out_shape=[
            jax.ShapeDtypeStruct((N, H), jnp.float32),
            jax.ShapeDtypeStruct((N, 1), jnp.float32),
        ],
    )(pa, pb, xn, nrm)


def _tc_out(pa, pb, xn, nrm, beta, w2t, b2):
    return pl.pallas_call(
        _tc_out_body,
        grid=(N // _ROWBLK,),
        in_specs=[
            pl.BlockSpec((_ROWBLK, GW), lambda i: (i, 0)),
            pl.BlockSpec((_ROWBLK, GW), lambda i: (i, 0)),
            pl.BlockSpec((_ROWBLK, H), lambda i: (i, 0)),
            pl.BlockSpec((_ROWBLK, 1), lambda i: (i, 0)),
            pl.BlockSpec((1, 1), lambda i: (0, 0)),
            pl.BlockSpec((H, C), lambda i: (0, 0)),
            pl.BlockSpec((1, C), lambda i: (0, 0)),
        ],
        out_specs=pl.BlockSpec((_ROWBLK, C), lambda i: (i, 0)),
        out_shape=jax.ShapeDtypeStruct((N, C), jnp.float32),
    )(pa, pb, xn, nrm, beta, w2t, b2)


def _sc_prop(xn, nrm, src2d, dst2d, bvec):
    mesh = plsc.VectorSubcoreMesh(
        core_axis_name="c", subcore_axis_name="s",
        num_cores=NC, num_subcores=NS)

    @functools.partial(
        pl.kernel,
        out_type=jax.ShapeDtypeStruct((NC, N, GW), jnp.float32),
        mesh=mesh,
        scratch_types=[
            pltpu.VMEM((NB, EB), jnp.int32),
            pltpu.VMEM((NB, EB), jnp.int32),
            pltpu.VMEM((N,), jnp.float32),
            pltpu.VMEM((EB, H), jnp.float32),
            pltpu.VMEM((EB, H), jnp.float32),
            pltpu.VMEM((EB, H), jnp.float32),
            pltpu.VMEM((EB, H), jnp.float32),
            pltpu.VMEM((EB, GW), jnp.float32),
            pltpu.VMEM((16,), jnp.float32),
            pltpu.VMEM((ZR, GW), jnp.float32),
            pltpu.VMEM_SHARED((N, GW), jnp.float32),
            pltpu.SemaphoreType.DMA,
            pltpu.SemaphoreType.DMA,
            pltpu.SemaphoreType.DMA,
            pltpu.SemaphoreType.DMA,
        ],
        compiler_params=pltpu.CompilerParams(
            needs_layout_passes=False, use_tc_tiling_on_sc=False),
    )
    def k(xn_hbm, nrm_hbm, src_hbm, dst_hbm, bvec_hbm, out_hbm,
          src_v, dst_v, nrm_v, gs0, gs1, gd0, gd1, ct_v, b_v, z_v, acc_sh,
          sgs0, sgs1, sgd0, sgd1):
        cid = lax.axis_index("c")
        sid = lax.axis_index("s")
        wid = cid * NS + sid

        zero16 = jnp.zeros((16,), jnp.float32)

        def zbody(i, carry):
            z_v[i, 0:16] = zero16
            z_v[i, 16:32] = zero16
            return carry

        lax.fori_loop(0, ZR, zbody, 0)
        pltpu.sync_copy(z_v, acc_sh.at[pl.ds(sid * ZR, ZR)])

        @pl.when(sid == NS - 1)
        def _():
            pltpu.sync_copy(z_v.at[pl.ds(0, ZTAIL)],
                            acc_sh.at[pl.ds(NS * ZR, ZTAIL)])

        pltpu.sync_copy(bvec_hbm, b_v)
        pltpu.sync_copy(nrm_hbm, nrm_v)
        pltpu.sync_copy(src_hbm.at[pl.ds(wid * NB, NB)], src_v)
        pltpu.sync_copy(dst_hbm.at[pl.ds(wid * NB, NB)], dst_v)
        plsc.subcore_barrier()

        bv16 = b_v[...]
        beta = bv16[0]
        ab = bv16[1]
        iota16 = lax.iota(jnp.int32, 16)
        diagcols = [(iota16 + kk) & 15 for kk in range(H)]
        ecols = iota16 + H
        slots = ((gs0, gd0, sgs0, sgd0), (gs1, gd1, sgs1, sgd1))

        for j in range(EB):
            ct_v[j, H:GW] = zero16

        def issue(bi, slot):
            gs, gd, sgs, sgd = slots[slot]
            pltpu.async_copy(xn_hbm.at[src_v.at[bi]], gs, sgs)
            pltpu.async_copy(xn_hbm.at[dst_v.at[bi]], gd, sgd)

        def wait_slot(slot):
            gs, gd, sgs, sgd = slots[slot]
            pltpu.make_async_copy(xn_hbm.at[pl.ds(0, EB)], gs, sgs).wait()
            pltpu.make_async_copy(xn_hbm.at[pl.ds(0, EB)], gd, sgd).wait()

        def compute_scatter(bi, slot):
            gs, gd, _, _ = slots[slot]
            for grp in range(EB // 16):
                rows = iota16 + (grp * 16)
                adiag = []
                terms = []
                for kk in range(H):
                    a = plsc.load_gather(gs, [rows, diagcols[kk]])
                    b = plsc.load_gather(gd, [rows, diagcols[kk]])
                    adiag.append(a)
                    terms.append(a * b)
                while len(terms) > 1:
                    terms = [terms[i] + terms[i + 1]
                             for i in range(0, len(terms), 2)]
                score = terms[0]
                s16 = src_v[bi, pl.ds(grp * 16, 16)]
                nsrc = plsc.load_gather(nrm_v, [s16])
                e16 = jnp.exp(beta * score - ab)
                sc16 = e16 * nsrc
                for kk in range(H):
                    plsc.store_scatter(ct_v, [rows, diagcols[kk]],
                                       adiag[kk] * sc16)
                plsc.store_scatter(ct_v, [rows, ecols], e16)
            pltpu.sync_copy(ct_v, acc_sh.at[dst_v.at[bi]], add=True)

        issue(0, 0)

        def pair(g, carry):
            b0 = 2 * g
            issue(b0 + 1, 1)
            wait_slot(0)
            compute_scatter(b0, 0)
            issue(b0 + 2, 0)
            wait_slot(1)
            compute_scatter(b0 + 1, 1)
            return carry

        lax.fori_loop(0, (NB - 1) // 2, pair, 0)
        wait_slot(0)
        compute_scatter(NB - 1, 0)

        plsc.subcore_barrier()
        pltpu.sync_copy(acc_sh.at[pl.ds(sid * ZR, ZR)],
                        out_hbm.at[cid, pl.ds(sid * ZR, ZR)])

        @pl.when(sid == NS - 1)
        def _():
            pltpu.sync_copy(acc_sh.at[pl.ds(NS * ZR, ZTAIL)],
                            out_hbm.at[cid, pl.ds(NS * ZR, ZTAIL)])

    return k(xn, nrm, src2d, dst2d, bvec)


def kernel(x, edge_index, W1, b1, W2, b2, beta2):
    x = x.astype(jnp.float32)
    ei = edge_index.astype(jnp.int32)
    src2d = ei[0].reshape(NW * NB, EB)
    dst2d = ei[1].reshape(NW * NB, EB)
    w1t = W1.astype(jnp.float32).T
    w2t = W2.astype(jnp.float32).T
    b1r = b1.astype(jnp.float32).reshape(1, H)
    b2r = b2.astype(jnp.float32).reshape(1, C)
    beta = beta2.astype(jnp.float32)[0]

    xn0, nrm0 = _tc_in(x, w1t, b1r)
    bvec1 = jnp.zeros((16,), jnp.float32).at[0].set(1.0).at[1].set(1.0)
    parts1 = _sc_prop(xn0, nrm0.reshape(N), src2d, dst2d, bvec1)
    xn1, nrm1 = _tc_mid(parts1[0], parts1[1], xn0, nrm0)
    bvec2 = jnp.zeros((16,), jnp.float32).at[0].set(beta).at[1].set(jnp.abs(beta))
    parts2 = _sc_prop(xn1, nrm1.reshape(N), src2d, dst2d, bvec2)
    return _tc_out(parts2[0], parts2[1], xn1, nrm1,
                   beta2.astype(jnp.float32).reshape(1, 1), w2t, b2r)
